# baseline (speedup 1.0000x reference)
import functools

import jax
import jax.numpy as jnp
from jax import lax
from jax.experimental import pallas as pl
from jax.experimental.pallas import tpu as pltpu
from jax.experimental.pallas import tpu_sc as plsc

_V = 1000000
_D = 32
_B = 16384
_F = 26
_NW = 32

_TC = 768
_NB_MAIN = 40
_NBLK = 1302
_TAIL = _V - _NBLK * _TC
_GC = 512
_NCH = (_B * _F) // (_NW * _GC)


def _cparams(sc_tiling):
    if sc_tiling:
        return pltpu.CompilerParams(
            use_tc_tiling_on_sc=False, needs_layout_passes=False)
    return pltpu.CompilerParams(needs_layout_passes=False)


@functools.lru_cache(maxsize=None)
def _make_transpose():
    info = plsc.get_sparse_core_info()
    nc = info.num_cores
    mesh = plsc.VectorSubcoreMesh(core_axis_name="c", subcore_axis_name="s")

    @functools.partial(
        pl.kernel,
        mesh=mesh,
        out_type=jax.ShapeDtypeStruct((_V * _D,), jnp.float32),
        scratch_types=[
            [pltpu.VMEM((_D, _TC), jnp.float32)] * 2,
            [pltpu.VMEM((_TC * _D,), jnp.float32)] * 2,
            [pltpu.SemaphoreType.DMA] * 2,
            [pltpu.SemaphoreType.DMA] * 2,
        ],
        compiler_params=_cparams(False),
    )
    def k(wt_hbm, tail_hbm, out_hbm, blks, outTs, rsems, wsems):
        wid = lax.axis_index("s") * nc + lax.axis_index("c")
        iota = lax.iota(jnp.int32, 16)
        kv0 = iota
        kv1 = iota + 16
        zeros = jnp.zeros((16,), jnp.int32)

        def blk_e0(t):
            return (wid * _NB_MAIN + t) * _TC

        def fire_read(t, b):
            pltpu.async_copy(
                wt_hbm.at[:, pl.ds(blk_e0(t), _TC)], blks[b], rsems[b])

        def drain_read(t, b):
            pltpu.make_async_copy(
                wt_hbm.at[:, pl.ds(blk_e0(t), _TC)], blks[b], rsems[b]).wait()

        def fire_write(t, b):
            pltpu.async_copy(
                outTs[b], out_hbm.at[pl.ds(blk_e0(t) * _D, _TC * _D)],
                wsems[b])

        def wait_write(t, b):
            pltpu.make_async_copy(
                outTs[b], out_hbm.at[pl.ds(blk_e0(t) * _D, _TC * _D)],
                wsems[b]).wait()

        def transpose(b):
            blk, outT = blks[b], outTs[b]

            def body(e, carry):
                ev = zeros + e
                outT[pl.ds(e * _D, 16)] = plsc.load_gather(blk, [kv0, ev])
                outT[pl.ds(e * _D + 16, 16)] = plsc.load_gather(blk, [kv1, ev])
                return carry

            lax.fori_loop(0, _TC, body, 0, unroll=4)

        fire_read(0, 0)
        fire_read(1, 1)

        def outer(t2, carry):
            for b in range(2):
                t = t2 * 2 + b
                drain_read(t, b)

                @pl.when(t2 > 0)
                def _():
                    wait_write(t - 2, b)

                transpose(b)
                fire_write(t, b)

                @pl.when(t + 2 < _NB_MAIN)
                def _():
                    fire_read(t + 2, b)

            return carry

        lax.fori_loop(0, _NB_MAIN // 2, outer, 0)

        wait_write(_NB_MAIN - 2, 0)
        wait_write(_NB_MAIN - 1, 1)

        @pl.when(wid < _NBLK - _NW * _NB_MAIN)
        def _():
            e0 = (_NW * _NB_MAIN + wid) * _TC
            pltpu.sync_copy(wt_hbm.at[:, pl.ds(e0, _TC)], blks[1])
            transpose(1)
            pltpu.sync_copy(outTs[1], out_hbm.at[pl.ds(e0 * _D, _TC * _D)])

        @pl.when(wid == _NBLK - _NW * _NB_MAIN)
        def _():
            pltpu.sync_copy(tail_hbm, outTs[0].at[pl.ds(0, _TAIL * _D)])
            pltpu.sync_copy(outTs[0].at[pl.ds(0, _TAIL * _D)],
                            out_hbm.at[pl.ds(_NBLK * _TC * _D, _TAIL * _D)])

    return k


@functools.lru_cache(maxsize=None)
def _make_gather():
    info = plsc.get_sparse_core_info()
    nc = info.num_cores
    mesh = plsc.VectorSubcoreMesh(core_axis_name="c", subcore_axis_name="s")
    b_per_w = _NCH * _GC

    @functools.partial(
        pl.kernel,
        mesh=mesh,
        out_type=jax.ShapeDtypeStruct((_F, _D, _B), jnp.float32),
        scratch_types=[
            pltpu.VMEM((b_per_w,), jnp.int32),
            [pltpu.VMEM((_GC, _D), jnp.float32)] * 2,
            [pltpu.VMEM((_D, _GC), jnp.float32)] * 2,
            pltpu.SemaphoreType.DMA,
            [pltpu.SemaphoreType.DMA] * 2,
            [pltpu.SemaphoreType.DMA] * 2,
        ],
        compiler_params=_cparams(True),
    )
    def k(idx_hbm, tab_hbm, out_hbm, idx_v, rows, rowsT, isem, gsems, wsems):
        wid = lax.axis_index("s") * nc + lax.axis_index("c")
        iota = lax.iota(jnp.int32, 16)
        zeros = jnp.zeros((16,), jnp.int32)

        def chunk_fj(t):
            c = wid + _NW * t
            return c // (_B // _GC), c % (_B // _GC)

        def idx_src(t):
            f, j = chunk_fj(t)
            return idx_hbm.at[pl.ds(f * _B + j * _GC, _GC)]

        for t in range(_NCH):
            pltpu.async_copy(idx_src(t), idx_v.at[pl.ds(t * _GC, _GC)], isem)
        for t in range(_NCH):
            pltpu.make_async_copy(
                idx_src(t), idx_v.at[pl.ds(t * _GC, _GC)], isem).wait()

        def fire_gather(t, b):
            pltpu.async_copy(
                tab_hbm.at[idx_v.at[pl.ds(t * _GC, _GC)]], rows[b], gsems[b])

        def drain_gather(t, b):
            pltpu.make_async_copy(
                tab_hbm.at[idx_v.at[pl.ds(t * _GC, _GC)]], rows[b],
                gsems[b]).wait()

        def out_dst(t):
            f, j = chunk_fj(t)
            return out_hbm.at[f, :, pl.ds(j * _GC, _GC)]

        def fire_write(t, b):
            pltpu.async_copy(rowsT[b], out_dst(t), wsems[b])

        def wait_write(t, b):
            pltpu.make_async_copy(rowsT[b], out_dst(t), wsems[b]).wait()

        def transpose(b):
            rv, rt = rows[b], rowsT[b]

            def kbody(kk, carry):
                kv = zeros + kk

                def body(i, c2):
                    b0 = i * 16
                    rt[kk, pl.ds(b0, 16)] = plsc.load_gather(
                        rv, [iota + b0, kv])
                    return c2

                lax.fori_loop(0, _GC // 16, body, 0, unroll=8)
                return carry

            lax.fori_loop(0, _D, kbody, 0)

        fire_gather(0, 0)
        fire_gather(1, 1)

        def outer(t2, carry):
            for b in range(2):
                t = t2 * 2 + b
                drain_gather(t, b)

                @pl.when(t2 > 0)
                def _():
                    wait_write(t - 2, b)

                transpose(b)
                fire_write(t, b)

                @pl.when(t2 < _NCH // 2 - 1)
                def _():
                    fire_gather(t + 2, b)

            return carry

        lax.fori_loop(0, _NCH // 2, outer, 0)

        wait_write(_NCH - 2, 0)
        wait_write(_NCH - 1, 1)

    return k


def kernel(x, weights):
    xt_flat = x.T.reshape(_B * _F).astype(jnp.int32)
    wt = weights.T
    tail = lax.slice(weights, (_NBLK * _TC, 0), (_V, _D)).reshape(_TAIL * _D)
    w_rm = _make_transpose()(wt, tail).reshape(_V, _D)
    o_t = _make_gather()(xt_flat, w_rm)
    return o_t.transpose(2, 0, 1)


# speedup vs baseline: 1.2184x; 1.2184x over previous
import functools

import jax
import jax.numpy as jnp
from jax import lax
from jax.experimental import pallas as pl
from jax.experimental.pallas import tpu as pltpu
from jax.experimental.pallas import tpu_sc as plsc

_V = 1000000
_D = 32
_B = 16384
_F = 26
_NW = 32

_TC = 768
_NB_MAIN = 40
_NBLK = 1302
_TAIL = _V - _NBLK * _TC
_GC = 512
_NCH = (_B * _F) // (_NW * _GC)


def _cparams(sc_tiling):
    if sc_tiling:
        return pltpu.CompilerParams(
            use_tc_tiling_on_sc=False, needs_layout_passes=False)
    return pltpu.CompilerParams(needs_layout_passes=False)


@functools.lru_cache(maxsize=None)
def _make_transpose():
    info = plsc.get_sparse_core_info()
    nc = info.num_cores
    mesh = plsc.VectorSubcoreMesh(core_axis_name="c", subcore_axis_name="s")

    @functools.partial(
        pl.kernel,
        mesh=mesh,
        out_type=jax.ShapeDtypeStruct((_V * _D,), jnp.float32),
        scratch_types=[
            [pltpu.VMEM((_D, _TC), jnp.float32)] * 2,
            [pltpu.VMEM((_TC * _D,), jnp.float32)] * 2,
            [pltpu.SemaphoreType.DMA] * 2,
            [pltpu.SemaphoreType.DMA] * 2,
        ],
        compiler_params=_cparams(False),
    )
    def k(wt_hbm, tail_hbm, out_hbm, blks, outTs, rsems, wsems):
        wid = lax.axis_index("s") * nc + lax.axis_index("c")
        iota = lax.iota(jnp.int32, 16)
        zeros = jnp.zeros((16,), jnp.int32)

        def blk_e0(t):
            return (wid * _NB_MAIN + t) * _TC

        def fire_read(t, b):
            pltpu.async_copy(
                wt_hbm.at[:, pl.ds(blk_e0(t), _TC)], blks[b], rsems[b])

        def drain_read(t, b):
            pltpu.make_async_copy(
                wt_hbm.at[:, pl.ds(blk_e0(t), _TC)], blks[b], rsems[b]).wait()

        def fire_write(t, b):
            pltpu.async_copy(
                outTs[b], out_hbm.at[pl.ds(blk_e0(t) * _D, _TC * _D)],
                wsems[b])

        def wait_write(t, b):
            pltpu.make_async_copy(
                outTs[b], out_hbm.at[pl.ds(blk_e0(t) * _D, _TC * _D)],
                wsems[b]).wait()

        iota32 = iota * _D

        def transpose(b):
            blk, outT = blks[b], outTs[b]

            def ebody(i, carry):
                e0 = i * 16

                def kbody(kk, c2):
                    plsc.store_scatter(
                        outT, [iota32 + (e0 * _D + kk)],
                        blk[kk, pl.ds(e0, 16)])
                    return c2

                lax.fori_loop(0, _D, kbody, 0, unroll=8)
                return carry

            lax.fori_loop(0, _TC // 16, ebody, 0)

        fire_read(0, 0)
        fire_read(1, 1)

        def outer(t2, carry):
            for b in range(2):
                t = t2 * 2 + b
                drain_read(t, b)

                @pl.when(t2 > 0)
                def _():
                    wait_write(t - 2, b)

                transpose(b)
                fire_write(t, b)

                @pl.when(t + 2 < _NB_MAIN)
                def _():
                    fire_read(t + 2, b)

            return carry

        lax.fori_loop(0, _NB_MAIN // 2, outer, 0)

        wait_write(_NB_MAIN - 2, 0)
        wait_write(_NB_MAIN - 1, 1)

        @pl.when(wid < _NBLK - _NW * _NB_MAIN)
        def _():
            e0 = (_NW * _NB_MAIN + wid) * _TC
            pltpu.sync_copy(wt_hbm.at[:, pl.ds(e0, _TC)], blks[1])
            transpose(1)
            pltpu.sync_copy(outTs[1], out_hbm.at[pl.ds(e0 * _D, _TC * _D)])

        @pl.when(wid == _NBLK - _NW * _NB_MAIN)
        def _():
            stage = outTs[0].at[pl.ds(0, _TAIL * _D)]
            pltpu.sync_copy(tail_hbm, stage)
            pltpu.sync_copy(
                stage, out_hbm.at[pl.ds(_NBLK * _TC * _D, _TAIL * _D)])

    return k


@functools.lru_cache(maxsize=None)
def _make_gather():
    info = plsc.get_sparse_core_info()
    nc = info.num_cores
    mesh = plsc.VectorSubcoreMesh(core_axis_name="c", subcore_axis_name="s")
    b_per_w = _NCH * _GC

    @functools.partial(
        pl.kernel,
        mesh=mesh,
        out_type=jax.ShapeDtypeStruct((_F, _D, _B), jnp.float32),
        scratch_types=[
            pltpu.VMEM((b_per_w,), jnp.int32),
            [pltpu.VMEM((_GC, _D), jnp.float32)] * 2,
            [pltpu.VMEM((_D, _GC), jnp.float32)] * 2,
            pltpu.SemaphoreType.DMA,
            [pltpu.SemaphoreType.DMA] * 2,
            [pltpu.SemaphoreType.DMA] * 2,
        ],
        compiler_params=_cparams(True),
    )
    def k(idx_hbm, tab_hbm, out_hbm, idx_v, rows, rowsT, isem, gsems, wsems):
        wid = lax.axis_index("s") * nc + lax.axis_index("c")
        iota = lax.iota(jnp.int32, 16)
        zeros = jnp.zeros((16,), jnp.int32)

        def chunk_fj(t):
            c = wid + _NW * t
            return c // (_B // _GC), c % (_B // _GC)

        def idx_src(t):
            f, j = chunk_fj(t)
            return idx_hbm.at[pl.ds(f * _B + j * _GC, _GC)]

        for t in range(_NCH):
            pltpu.async_copy(idx_src(t), idx_v.at[pl.ds(t * _GC, _GC)], isem)
        for t in range(_NCH):
            pltpu.make_async_copy(
                idx_src(t), idx_v.at[pl.ds(t * _GC, _GC)], isem).wait()

        def fire_gather(t, b):
            pltpu.async_copy(
                tab_hbm.at[idx_v.at[pl.ds(t * _GC, _GC)]], rows[b], gsems[b])

        def drain_gather(t, b):
            pltpu.make_async_copy(
                tab_hbm.at[idx_v.at[pl.ds(t * _GC, _GC)]], rows[b],
                gsems[b]).wait()

        def out_dst(t):
            f, j = chunk_fj(t)
            return out_hbm.at[f, :, pl.ds(j * _GC, _GC)]

        def fire_write(t, b):
            pltpu.async_copy(rowsT[b], out_dst(t), wsems[b])

        def wait_write(t, b):
            pltpu.make_async_copy(rowsT[b], out_dst(t), wsems[b]).wait()

        kv0 = iota
        kv1 = iota + 16

        def transpose(b):
            rv, rt = rows[b], rowsT[b]

            def body(e, carry):
                ev = zeros + e
                plsc.store_scatter(rt, [kv0, ev], rv[e, pl.ds(0, 16)])
                plsc.store_scatter(rt, [kv1, ev], rv[e, pl.ds(16, 16)])
                return carry

            lax.fori_loop(0, _GC, body, 0, unroll=8)

        fire_gather(0, 0)
        fire_gather(1, 1)

        def outer(t2, carry):
            for b in range(2):
                t = t2 * 2 + b
                drain_gather(t, b)

                @pl.when(t2 > 0)
                def _():
                    wait_write(t - 2, b)

                transpose(b)
                fire_write(t, b)

                @pl.when(t2 < _NCH // 2 - 1)
                def _():
                    fire_gather(t + 2, b)

            return carry

        lax.fori_loop(0, _NCH // 2, outer, 0)

        wait_write(_NCH - 2, 0)
        wait_write(_NCH - 1, 1)

    return k


def kernel(x, weights):
    xt_flat = x.T.reshape(_B * _F).astype(jnp.int32)
    wt = weights.T
    tail = lax.slice(weights, (_NBLK * _TC, 0), (_V, _D)).reshape(_TAIL * _D)
    w_rm = _make_transpose()(wt, tail).reshape(_V, _D)
    o_t = _make_gather()(xt_flat, w_rm)
    return o_t.transpose(2, 0, 1)


# speedup vs baseline: 5.6131x; 4.6070x over previous
import functools

import jax
import jax.numpy as jnp
from jax import lax
from jax.experimental import pallas as pl
from jax.experimental.pallas import tpu as pltpu
from jax.experimental.pallas import tpu_sc as plsc

_V = 1000000
_D = 32
_B = 16384
_F = 26
_NW = 32

_TC = 768
_NB_MAIN = 40
_NBLK = 1302
_TAIL = _V - _NBLK * _TC
_GC = 512
_NCH = (_B * _F) // (_NW * _GC)


def _cparams(sc_tiling):
    if sc_tiling:
        return pltpu.CompilerParams(
            use_tc_tiling_on_sc=False, needs_layout_passes=False)
    return pltpu.CompilerParams(needs_layout_passes=False)


@functools.lru_cache(maxsize=None)
def _make_transpose():
    info = plsc.get_sparse_core_info()
    nc = info.num_cores
    mesh = plsc.VectorSubcoreMesh(core_axis_name="c", subcore_axis_name="s")

    @functools.partial(
        pl.kernel,
        mesh=mesh,
        out_type=jax.ShapeDtypeStruct((_V * _D,), jnp.float32),
        scratch_types=[
            [pltpu.VMEM((_D, _TC), jnp.float32)] * 2,
            [pltpu.VMEM((_TC * _D,), jnp.float32)] * 2,
            [pltpu.SemaphoreType.DMA] * 2,
            [pltpu.SemaphoreType.DMA] * 2,
        ],
        compiler_params=_cparams(False),
    )
    def k(wt_hbm, tail_hbm, out_hbm, blks, outTs, rsems, wsems):
        wid = lax.axis_index("s") * nc + lax.axis_index("c")
        iota = lax.iota(jnp.int32, 16)
        zeros = jnp.zeros((16,), jnp.int32)

        def blk_e0(t):
            return (wid * _NB_MAIN + t) * _TC

        def fire_read(t, b):
            pltpu.async_copy(
                wt_hbm.at[:, pl.ds(blk_e0(t), _TC)], blks[b], rsems[b])

        def drain_read(t, b):
            pltpu.make_async_copy(
                wt_hbm.at[:, pl.ds(blk_e0(t), _TC)], blks[b], rsems[b]).wait()

        def fire_write(t, b):
            pltpu.async_copy(
                outTs[b], out_hbm.at[pl.ds(blk_e0(t) * _D, _TC * _D)],
                wsems[b])

        def wait_write(t, b):
            pltpu.make_async_copy(
                outTs[b], out_hbm.at[pl.ds(blk_e0(t) * _D, _TC * _D)],
                wsems[b]).wait()

        iota32 = iota * _D

        def transpose(b):
            blk, outT = blks[b], outTs[b]

            def ebody(i, carry):
                e0 = i * 16

                @functools.partial(
                    plsc.parallel_loop, 0, _D, unroll=8,
                    carry=iota32 + e0 * _D)
                def kbody(kk, addr):
                    plsc.store_scatter(outT, [addr], blk[kk, pl.ds(e0, 16)])
                    return addr + 1

                return carry

            lax.fori_loop(0, _TC // 16, ebody, 0)

        fire_read(0, 0)
        fire_read(1, 1)

        def outer(t2, carry):
            for b in range(2):
                t = t2 * 2 + b
                drain_read(t, b)

                @pl.when(t2 > 0)
                def _():
                    wait_write(t - 2, b)

                transpose(b)
                fire_write(t, b)

                @pl.when(t + 2 < _NB_MAIN)
                def _():
                    fire_read(t + 2, b)

            return carry

        lax.fori_loop(0, _NB_MAIN // 2, outer, 0)

        wait_write(_NB_MAIN - 2, 0)
        wait_write(_NB_MAIN - 1, 1)

        @pl.when(wid < _NBLK - _NW * _NB_MAIN)
        def _():
            e0 = (_NW * _NB_MAIN + wid) * _TC
            pltpu.sync_copy(wt_hbm.at[:, pl.ds(e0, _TC)], blks[1])
            transpose(1)
            pltpu.sync_copy(outTs[1], out_hbm.at[pl.ds(e0 * _D, _TC * _D)])

        @pl.when(wid == _NBLK - _NW * _NB_MAIN)
        def _():
            stage = outTs[0].at[pl.ds(0, _TAIL * _D)]
            pltpu.sync_copy(tail_hbm, stage)
            pltpu.sync_copy(
                stage, out_hbm.at[pl.ds(_NBLK * _TC * _D, _TAIL * _D)])

    return k


@functools.lru_cache(maxsize=None)
def _make_gather():
    info = plsc.get_sparse_core_info()
    nc = info.num_cores
    mesh = plsc.VectorSubcoreMesh(core_axis_name="c", subcore_axis_name="s")
    b_per_w = _NCH * _GC

    @functools.partial(
        pl.kernel,
        mesh=mesh,
        out_type=jax.ShapeDtypeStruct((_F, _D, _B), jnp.float32),
        scratch_types=[
            pltpu.VMEM((b_per_w,), jnp.int32),
            [pltpu.VMEM((_GC, _D), jnp.float32)] * 2,
            [pltpu.VMEM((_D, _GC), jnp.float32)] * 2,
            pltpu.SemaphoreType.DMA,
            [pltpu.SemaphoreType.DMA] * 2,
            [pltpu.SemaphoreType.DMA] * 2,
        ],
        compiler_params=_cparams(True),
    )
    def k(idx_hbm, tab_hbm, out_hbm, idx_v, rows, rowsT, isem, gsems, wsems):
        wid = lax.axis_index("s") * nc + lax.axis_index("c")
        iota = lax.iota(jnp.int32, 16)
        zeros = jnp.zeros((16,), jnp.int32)

        def chunk_fj(t):
            c = wid + _NW * t
            return c // (_B // _GC), c % (_B // _GC)

        def idx_src(t):
            f, j = chunk_fj(t)
            return idx_hbm.at[pl.ds(f * _B + j * _GC, _GC)]

        for t in range(_NCH):
            pltpu.async_copy(idx_src(t), idx_v.at[pl.ds(t * _GC, _GC)], isem)
        for t in range(_NCH):
            pltpu.make_async_copy(
                idx_src(t), idx_v.at[pl.ds(t * _GC, _GC)], isem).wait()

        def fire_gather(t, b):
            pltpu.async_copy(
                tab_hbm.at[idx_v.at[pl.ds(t * _GC, _GC)]], rows[b], gsems[b])

        def drain_gather(t, b):
            pltpu.make_async_copy(
                tab_hbm.at[idx_v.at[pl.ds(t * _GC, _GC)]], rows[b],
                gsems[b]).wait()

        def out_dst(t):
            f, j = chunk_fj(t)
            return out_hbm.at[f, :, pl.ds(j * _GC, _GC)]

        def fire_write(t, b):
            pltpu.async_copy(rowsT[b], out_dst(t), wsems[b])

        def wait_write(t, b):
            pltpu.make_async_copy(rowsT[b], out_dst(t), wsems[b]).wait()

        kv0 = iota
        kv1 = iota + 16

        def transpose(b):
            rv, rt = rows[b], rowsT[b]

            @functools.partial(
                plsc.parallel_loop, 0, _GC, unroll=8, carry=zeros)
            def body(e, ev):
                plsc.store_scatter(rt, [kv0, ev], rv[e, pl.ds(0, 16)])
                plsc.store_scatter(rt, [kv1, ev], rv[e, pl.ds(16, 16)])
                return ev + 1

        fire_gather(0, 0)
        fire_gather(1, 1)

        def outer(t2, carry):
            for b in range(2):
                t = t2 * 2 + b
                drain_gather(t, b)

                @pl.when(t2 > 0)
                def _():
                    wait_write(t - 2, b)

                transpose(b)
                fire_write(t, b)

                @pl.when(t2 < _NCH // 2 - 1)
                def _():
                    fire_gather(t + 2, b)

            return carry

        lax.fori_loop(0, _NCH // 2, outer, 0)

        wait_write(_NCH - 2, 0)
        wait_write(_NCH - 1, 1)

    return k


def kernel(x, weights):
    xt_flat = x.T.reshape(_B * _F).astype(jnp.int32)
    wt = weights.T
    tail = lax.slice(weights, (_NBLK * _TC, 0), (_V, _D)).reshape(_TAIL * _D)
    w_rm = _make_transpose()(wt, tail).reshape(_V, _D)
    o_t = _make_gather()(xt_flat, w_rm)
    return o_t.transpose(2, 0, 1)
